# baseline (speedup 1.0000x reference)
import functools

import jax
import jax.numpy as jnp
from jax import lax
from jax.experimental import pallas as pl
from jax.experimental.pallas import tpu as pltpu
from jax.experimental.pallas import tpu_sc as plsc

N_NODES = 10000
N_EDGES = 320000
D_FEAT = 128

NC = 2
NS = 16
NW = NC * NS

N_PAD = 10240
ROWS_PER_TILE = N_PAD // NS
ZCHUNK = 128

CHUNK = 80
K_PER_TILE = N_EDGES // NW // CHUNK
EPT = K_PER_TILE * CHUNK
K_HALF = 63

_mesh = plsc.VectorSubcoreMesh(core_axis_name="c", subcore_axis_name="s")


def _fill_1d(ref, n, value):
    def body(i, _):
        ref[pl.ds(i * 16, 16)] = jnp.full((16,), value, dtype=jnp.float32)
        return ()
    lax.fori_loop(0, n // 16, body, ())


def _stage(flat_ref, base, buf):
    for m in range(CHUNK // 16):
        buf[pl.ds(16 * m, 16)] = flat_ref[pl.ds(base + 16 * m, 16)]


def _deg_body(src_hbm, dst_hbm, h_hbm,
              idx_s, idx_d, st_s0, st_s1, st_s2, st_d0, st_d1, st_d2,
              ones_v, zeros_v, hs_sh, hd_sh, sem):
    cid = lax.axis_index("c")
    sid = lax.axis_index("s")
    wid = sid * NC + cid

    pltpu.sync_copy(src_hbm.at[pl.ds(wid * EPT, EPT)], idx_s)
    pltpu.sync_copy(dst_hbm.at[pl.ds(wid * EPT, EPT)], idx_d)
    _fill_1d(ones_v, CHUNK, 1.0)
    _fill_1d(zeros_v, ROWS_PER_TILE, 0.0)
    pltpu.sync_copy(zeros_v, hs_sh.at[pl.ds(sid * ROWS_PER_TILE, ROWS_PER_TILE)])
    pltpu.sync_copy(zeros_v, hd_sh.at[pl.ds(sid * ROWS_PER_TILE, ROWS_PER_TILE)])
    plsc.subcore_barrier()

    def fire(i, st_s, st_d):
        _stage(idx_s, i * CHUNK, st_s)
        _stage(idx_d, i * CHUNK, st_d)
        pltpu.async_copy(ones_v, hs_sh.at[st_s], sem, add=True)
        pltpu.async_copy(ones_v, hd_sh.at[st_d], sem, add=True)

    def drain():
        pltpu.make_async_copy(ones_v, hs_sh.at[st_s0], sem).wait()
        pltpu.make_async_copy(ones_v, hd_sh.at[st_d0], sem).wait()

    def role(i, st_s, st_d):
        @pl.when(i < K_PER_TILE)
        def _():
            fire(i, st_s, st_d)

            @pl.when(i >= 2)
            def _():
                drain()

    def body(k3, _):
        i = 3 * k3
        role(i, st_s0, st_d0)
        role(i + 1, st_s1, st_d1)
        role(i + 2, st_s2, st_d2)
        return ()

    lax.fori_loop(0, (K_PER_TILE + 2) // 3, body, ())
    drain()
    drain()
    plsc.subcore_barrier()

    sl = pl.ds(sid * ROWS_PER_TILE, ROWS_PER_TILE)
    pltpu.sync_copy(hs_sh.at[sl], h_hbm.at[0, cid, sl])
    pltpu.sync_copy(hd_sh.at[sl], h_hbm.at[1, cid, sl])


_deg_call = pl.kernel(
    _deg_body,
    out_type=jax.ShapeDtypeStruct((2, NC, N_PAD), jnp.float32),
    mesh=_mesh,
    scratch_types=[
        pltpu.VMEM((EPT,), jnp.int32),
        pltpu.VMEM((EPT,), jnp.int32),
        pltpu.VMEM((CHUNK,), jnp.int32),
        pltpu.VMEM((CHUNK,), jnp.int32),
        pltpu.VMEM((CHUNK,), jnp.int32),
        pltpu.VMEM((CHUNK,), jnp.int32),
        pltpu.VMEM((CHUNK,), jnp.int32),
        pltpu.VMEM((CHUNK,), jnp.int32),
        pltpu.VMEM((CHUNK,), jnp.float32),
        pltpu.VMEM((ROWS_PER_TILE,), jnp.float32),
        pltpu.VMEM_SHARED((N_PAD,), jnp.float32),
        pltpu.VMEM_SHARED((N_PAD,), jnp.float32),
        pltpu.SemaphoreType.DMA,
    ],
)


def _main_body(src_hbm, dst_hbm, y_hbm, part_hbm,
               idx_s, idx_d, st_d0, st_d1, st_d2,
               rows0, rows1, rows2, acc_sh,
               g0, g1, g2, s0, s1, s2):
    cid = lax.axis_index("c")
    sid = lax.axis_index("s")
    wid = sid * NC + cid

    pre_s = pltpu.async_copy(src_hbm.at[pl.ds(wid * EPT, EPT)], idx_s, g0)
    pre_d = pltpu.async_copy(
        dst_hbm.at[pl.ds(wid * EPT, K_HALF * CHUNK)], idx_d, g1)

    def sidx(k):
        return idx_s.at[pl.ds(k * CHUNK, CHUNK)]

    def zrow(i, _):
        def zlane(j, _):
            rows0[i, pl.ds(j * 16, 16)] = jnp.zeros((16,), jnp.float32)
            return ()
        lax.fori_loop(0, D_FEAT // 16, zlane, ())
        return ()
    lax.fori_loop(0, CHUNK, zrow, ())

    def zacc(k, _):
        pltpu.sync_copy(
            rows0,
            acc_sh.at[pl.ds(sid * ROWS_PER_TILE + k * CHUNK, CHUNK)])
        return ()
    pre_s.wait()
    pre_d.wait()
    lax.fori_loop(0, ROWS_PER_TILE // CHUNK, zacc, ())
    plsc.subcore_barrier()

    def role(i, rows, gsem, ssem, prev_ssem, st_d, rows_nn, gsem_nn):
        @pl.when(i < K_PER_TILE)
        def _():
            @pl.when(i == K_HALF)
            def _():
                pltpu.sync_copy(
                    dst_hbm.at[pl.ds(wid * EPT + K_HALF * CHUNK,
                                     (K_PER_TILE - K_HALF) * CHUNK)],
                    idx_d.at[pl.ds(0, (K_PER_TILE - K_HALF) * CHUNK)])

            pltpu.make_async_copy(y_hbm.at[sidx(i)], rows, gsem).wait()
            base = jnp.where(i < K_HALF, i, i - K_HALF) * CHUNK
            _stage(idx_d, base, st_d)
            pltpu.async_copy(rows, acc_sh.at[st_d], ssem, add=True)

            @pl.when(i >= 1)
            def _():
                pltpu.make_async_copy(rows, acc_sh.at[st_d], prev_ssem).wait()

            @pl.when(i + 2 < K_PER_TILE)
            def _():
                pltpu.async_copy(y_hbm.at[sidx(i + 2)], rows_nn, gsem_nn)

    pltpu.async_copy(y_hbm.at[sidx(0)], rows0, g0)
    pltpu.async_copy(y_hbm.at[sidx(1)], rows1, g1)

    def body(k3, _):
        i = 3 * k3
        role(i, rows0, g0, s0, s2, st_d0, rows2, g2)
        role(i + 1, rows1, g1, s1, s0, st_d1, rows0, g0)
        role(i + 2, rows2, g2, s2, s1, st_d2, rows1, g1)
        return ()

    lax.fori_loop(0, (K_PER_TILE + 2) // 3, body, ())
    pltpu.make_async_copy(rows1, acc_sh.at[st_d1], s1).wait()
    plsc.subcore_barrier()

    def wb(k, _):
        sl = pl.ds(sid * ROWS_PER_TILE + k * ZCHUNK, ZCHUNK)
        pltpu.sync_copy(acc_sh.at[sl], part_hbm.at[cid, sl])
        return ()
    lax.fori_loop(0, ROWS_PER_TILE // ZCHUNK, wb, ())


_main_call = pl.kernel(
    _main_body,
    out_type=jax.ShapeDtypeStruct((NC, N_PAD, D_FEAT), jnp.float32),
    mesh=_mesh,
    scratch_types=[
        pltpu.VMEM((EPT,), jnp.int32),
        pltpu.VMEM((K_HALF * CHUNK,), jnp.int32),
        pltpu.VMEM((CHUNK,), jnp.int32),
        pltpu.VMEM((CHUNK,), jnp.int32),
        pltpu.VMEM((CHUNK,), jnp.int32),
        pltpu.VMEM((CHUNK, D_FEAT), jnp.float32),
        pltpu.VMEM((CHUNK, D_FEAT), jnp.float32),
        pltpu.VMEM((CHUNK, D_FEAT), jnp.float32),
        pltpu.VMEM_SHARED((N_PAD, D_FEAT), jnp.float32),
        pltpu.SemaphoreType.DMA,
        pltpu.SemaphoreType.DMA,
        pltpu.SemaphoreType.DMA,
        pltpu.SemaphoreType.DMA,
        pltpu.SemaphoreType.DMA,
        pltpu.SemaphoreType.DMA,
    ],
)


def _prescale_body(x_ref, hs_ref, y_ref):
    deg = hs_ref[0, 0] + hs_ref[0, 1]
    norm = jax.lax.rsqrt(jnp.clip(deg, 1.0, None))
    y_ref[...] = x_ref[...] * norm


def _final_body(part_ref, hd_ref, out_ref):
    deg = hd_ref[0, 0] + hd_ref[0, 1]
    norm = jax.lax.rsqrt(jnp.clip(deg, 1.0, None))
    out_ref[...] = (part_ref[0] + part_ref[1]) * norm


_BLK = 1000


def _prescale_call(x, hs3):
    return pl.pallas_call(
        _prescale_body,
        grid=(N_NODES // _BLK,),
        in_specs=[
            pl.BlockSpec((_BLK, D_FEAT), lambda i: (i, 0)),
            pl.BlockSpec((1, NC, _BLK, 1), lambda i: (0, 0, i, 0)),
        ],
        out_specs=pl.BlockSpec((_BLK, D_FEAT), lambda i: (i, 0)),
        out_shape=jax.ShapeDtypeStruct((N_NODES, D_FEAT), jnp.float32),
    )(x, hs3)


def _final_call(parts, hd3):
    return pl.pallas_call(
        _final_body,
        grid=(N_NODES // _BLK,),
        in_specs=[
            pl.BlockSpec((NC, _BLK, D_FEAT), lambda i: (0, i, 0)),
            pl.BlockSpec((1, NC, _BLK, 1), lambda i: (1, 0, i, 0)),
        ],
        out_specs=pl.BlockSpec((_BLK, D_FEAT), lambda i: (i, 0)),
        out_shape=jax.ShapeDtypeStruct((N_NODES, D_FEAT), jnp.float32),
    )(parts, hd3)


def kernel(x, edge_index):
    src_flat = edge_index[0].astype(jnp.int32)
    dst_flat = edge_index[1].astype(jnp.int32)
    h = _deg_call(src_flat, dst_flat).reshape(2, NC, N_PAD, 1)
    y = _prescale_call(x, h)
    parts = _main_call(src_flat, dst_flat, y)
    return _final_call(parts, h)


# speedup vs baseline: 1.0142x; 1.0142x over previous
import functools

import jax
import jax.numpy as jnp
from jax import lax
from jax.experimental import pallas as pl
from jax.experimental.pallas import tpu as pltpu
from jax.experimental.pallas import tpu_sc as plsc

N_NODES = 10000
N_EDGES = 320000
D_FEAT = 128

NC = 2
NS = 16
NW = NC * NS

N_PAD = 10240
ROWS_PER_TILE = N_PAD // NS
ZCHUNK = 128

CHUNK = 80
K_PER_TILE = N_EDGES // NW // CHUNK
EPT = K_PER_TILE * CHUNK
K_HALF = 63

_mesh = plsc.VectorSubcoreMesh(core_axis_name="c", subcore_axis_name="s")


def _fill_1d(ref, n, value):
    def body(i, _):
        ref[pl.ds(i * 16, 16)] = jnp.full((16,), value, dtype=jnp.float32)
        return ()
    lax.fori_loop(0, n // 16, body, ())


def _stage(flat_ref, base, buf):
    for m in range(CHUNK // 16):
        buf[pl.ds(16 * m, 16)] = flat_ref[pl.ds(base + 16 * m, 16)]


def _deg_body(src_hbm, dst_hbm, hsrc_hbm, hdst_hbm,
              idx_s, idx_d, st_s0, st_s1, st_s2, st_d0, st_d1, st_d2,
              ones_v, zeros_v, hs_sh, hd_sh, sem):
    cid = lax.axis_index("c")
    sid = lax.axis_index("s")
    wid = sid * NC + cid

    pltpu.sync_copy(src_hbm.at[pl.ds(wid * EPT, EPT)], idx_s)
    pltpu.sync_copy(dst_hbm.at[pl.ds(wid * EPT, EPT)], idx_d)
    _fill_1d(ones_v, CHUNK, 1.0)
    _fill_1d(zeros_v, ROWS_PER_TILE, 0.0)
    pltpu.sync_copy(zeros_v, hs_sh.at[pl.ds(sid * ROWS_PER_TILE, ROWS_PER_TILE)])
    pltpu.sync_copy(zeros_v, hd_sh.at[pl.ds(sid * ROWS_PER_TILE, ROWS_PER_TILE)])
    plsc.subcore_barrier()

    def fire(i, st_s, st_d):
        _stage(idx_s, i * CHUNK, st_s)
        _stage(idx_d, i * CHUNK, st_d)
        pltpu.async_copy(ones_v, hs_sh.at[st_s], sem, add=True)
        pltpu.async_copy(ones_v, hd_sh.at[st_d], sem, add=True)

    def drain():
        pltpu.make_async_copy(ones_v, hs_sh.at[st_s0], sem).wait()
        pltpu.make_async_copy(ones_v, hd_sh.at[st_d0], sem).wait()

    def role(i, st_s, st_d):
        @pl.when(i < K_PER_TILE)
        def _():
            fire(i, st_s, st_d)

            @pl.when(i >= 2)
            def _():
                drain()

    def body(k3, _):
        i = 3 * k3
        role(i, st_s0, st_d0)
        role(i + 1, st_s1, st_d1)
        role(i + 2, st_s2, st_d2)
        return ()

    lax.fori_loop(0, (K_PER_TILE + 2) // 3, body, ())
    drain()
    drain()
    plsc.subcore_barrier()

    sl = pl.ds(sid * ROWS_PER_TILE, ROWS_PER_TILE)
    pltpu.sync_copy(hs_sh.at[sl], hsrc_hbm.at[cid, sl])
    pltpu.sync_copy(hd_sh.at[sl], hdst_hbm.at[cid, sl])


_deg_call = pl.kernel(
    _deg_body,
    out_type=(
        jax.ShapeDtypeStruct((NC, N_PAD), jnp.float32),
        jax.ShapeDtypeStruct((NC, N_PAD), jnp.float32),
    ),
    mesh=_mesh,
    scratch_types=[
        pltpu.VMEM((EPT,), jnp.int32),
        pltpu.VMEM((EPT,), jnp.int32),
        pltpu.VMEM((CHUNK,), jnp.int32),
        pltpu.VMEM((CHUNK,), jnp.int32),
        pltpu.VMEM((CHUNK,), jnp.int32),
        pltpu.VMEM((CHUNK,), jnp.int32),
        pltpu.VMEM((CHUNK,), jnp.int32),
        pltpu.VMEM((CHUNK,), jnp.int32),
        pltpu.VMEM((CHUNK,), jnp.float32),
        pltpu.VMEM((ROWS_PER_TILE,), jnp.float32),
        pltpu.VMEM_SHARED((N_PAD,), jnp.float32),
        pltpu.VMEM_SHARED((N_PAD,), jnp.float32),
        pltpu.SemaphoreType.DMA,
    ],
)


def _main_body(src_hbm, dst_hbm, y_hbm, part_hbm,
               idx_s, idx_d, st_d0, st_d1, st_d2,
               rows0, rows1, rows2, acc_sh,
               g0, g1, g2, s0, s1, s2):
    cid = lax.axis_index("c")
    sid = lax.axis_index("s")
    wid = sid * NC + cid

    pltpu.sync_copy(src_hbm.at[pl.ds(wid * EPT, EPT)], idx_s)
    pltpu.sync_copy(dst_hbm.at[pl.ds(wid * EPT, K_HALF * CHUNK)], idx_d)

    def sidx(k):
        return idx_s.at[pl.ds(k * CHUNK, CHUNK)]

    def zrow(i, _):
        def zlane(j, _):
            rows0[i, pl.ds(j * 16, 16)] = jnp.zeros((16,), jnp.float32)
            return ()
        lax.fori_loop(0, D_FEAT // 16, zlane, ())
        return ()
    lax.fori_loop(0, CHUNK, zrow, ())

    def zacc(k, _):
        pltpu.sync_copy(
            rows0,
            acc_sh.at[pl.ds(sid * ROWS_PER_TILE + k * CHUNK, CHUNK)])
        return ()
    lax.fori_loop(0, ROWS_PER_TILE // CHUNK, zacc, ())
    plsc.subcore_barrier()

    def role(i, rows, gsem, ssem, prev_ssem, st_d, rows_nn, gsem_nn):
        @pl.when(i < K_PER_TILE)
        def _():
            @pl.when(i == K_HALF)
            def _():
                pltpu.sync_copy(
                    dst_hbm.at[pl.ds(wid * EPT + K_HALF * CHUNK,
                                     (K_PER_TILE - K_HALF) * CHUNK)],
                    idx_d.at[pl.ds(0, (K_PER_TILE - K_HALF) * CHUNK)])

            pltpu.make_async_copy(y_hbm.at[sidx(i)], rows, gsem).wait()
            base = jnp.where(i < K_HALF, i, i - K_HALF) * CHUNK
            _stage(idx_d, base, st_d)
            pltpu.async_copy(rows, acc_sh.at[st_d], ssem, add=True)

            @pl.when(i >= 1)
            def _():
                pltpu.make_async_copy(rows, acc_sh.at[st_d], prev_ssem).wait()

            @pl.when(i + 2 < K_PER_TILE)
            def _():
                pltpu.async_copy(y_hbm.at[sidx(i + 2)], rows_nn, gsem_nn)

    pltpu.async_copy(y_hbm.at[sidx(0)], rows0, g0)
    pltpu.async_copy(y_hbm.at[sidx(1)], rows1, g1)

    def body(k3, _):
        i = 3 * k3
        role(i, rows0, g0, s0, s2, st_d0, rows2, g2)
        role(i + 1, rows1, g1, s1, s0, st_d1, rows0, g0)
        role(i + 2, rows2, g2, s2, s1, st_d2, rows1, g1)
        return ()

    lax.fori_loop(0, (K_PER_TILE + 2) // 3, body, ())
    pltpu.make_async_copy(rows1, acc_sh.at[st_d1], s1).wait()
    plsc.subcore_barrier()

    def wb(k, _):
        sl = pl.ds(sid * ROWS_PER_TILE + k * ZCHUNK, ZCHUNK)
        pltpu.sync_copy(acc_sh.at[sl], part_hbm.at[cid, sl])
        return ()
    lax.fori_loop(0, ROWS_PER_TILE // ZCHUNK, wb, ())


_main_call = pl.kernel(
    _main_body,
    out_type=jax.ShapeDtypeStruct((NC, N_PAD, D_FEAT), jnp.float32),
    mesh=_mesh,
    scratch_types=[
        pltpu.VMEM((EPT,), jnp.int32),
        pltpu.VMEM((K_HALF * CHUNK,), jnp.int32),
        pltpu.VMEM((CHUNK,), jnp.int32),
        pltpu.VMEM((CHUNK,), jnp.int32),
        pltpu.VMEM((CHUNK,), jnp.int32),
        pltpu.VMEM((CHUNK, D_FEAT), jnp.float32),
        pltpu.VMEM((CHUNK, D_FEAT), jnp.float32),
        pltpu.VMEM((CHUNK, D_FEAT), jnp.float32),
        pltpu.VMEM_SHARED((N_PAD, D_FEAT), jnp.float32),
        pltpu.SemaphoreType.DMA,
        pltpu.SemaphoreType.DMA,
        pltpu.SemaphoreType.DMA,
        pltpu.SemaphoreType.DMA,
        pltpu.SemaphoreType.DMA,
        pltpu.SemaphoreType.DMA,
    ],
)


def _prescale_body(x_ref, hs_ref, y_ref):
    deg = hs_ref[0] + hs_ref[1]
    norm = jax.lax.rsqrt(jnp.clip(deg, 1.0, None))
    y_ref[...] = x_ref[...] * norm


def _final_body(part_ref, hd_ref, out_ref):
    deg = hd_ref[0] + hd_ref[1]
    norm = jax.lax.rsqrt(jnp.clip(deg, 1.0, None))
    out_ref[...] = (part_ref[0] + part_ref[1]) * norm


_BLK = 1000


def _prescale_call(x, hs3):
    return pl.pallas_call(
        _prescale_body,
        grid=(N_NODES // _BLK,),
        in_specs=[
            pl.BlockSpec((_BLK, D_FEAT), lambda i: (i, 0)),
            pl.BlockSpec((NC, _BLK, 1), lambda i: (0, i, 0)),
        ],
        out_specs=pl.BlockSpec((_BLK, D_FEAT), lambda i: (i, 0)),
        out_shape=jax.ShapeDtypeStruct((N_NODES, D_FEAT), jnp.float32),
    )(x, hs3)


def _final_call(parts, hd3):
    return pl.pallas_call(
        _final_body,
        grid=(N_NODES // _BLK,),
        in_specs=[
            pl.BlockSpec((NC, _BLK, D_FEAT), lambda i: (0, i, 0)),
            pl.BlockSpec((NC, _BLK, 1), lambda i: (0, i, 0)),
        ],
        out_specs=pl.BlockSpec((_BLK, D_FEAT), lambda i: (i, 0)),
        out_shape=jax.ShapeDtypeStruct((N_NODES, D_FEAT), jnp.float32),
    )(parts, hd3)


def kernel(x, edge_index):
    src_flat = edge_index[0].astype(jnp.int32)
    dst_flat = edge_index[1].astype(jnp.int32)
    hsrc, hdst = _deg_call(src_flat, dst_flat)
    y = _prescale_call(x, hsrc.reshape(NC, N_PAD, 1))
    parts = _main_call(src_flat, dst_flat, y)
    return _final_call(parts, hdst.reshape(NC, N_PAD, 1))
